# Initial kernel scaffold; baseline (speedup 1.0000x reference)
import jax
import jax.numpy as jnp
from jax.experimental import pallas as pl


def kernel(x, edge_index, edge_attr, batch, params):
    raise NotImplementedError("write your pallas kernel here")


# speedup vs baseline: 1.5078x; 1.5078x over previous
import functools

import jax
import jax.numpy as jnp
from jax import lax
from jax.experimental import pallas as pl
from jax.experimental.pallas import tpu as pltpu
from jax.experimental.pallas import tpu_sc as plsc

N = 50000
E = 800000
DIN = 128
DE = 16
H = 64
G = 64
EPS = 1e-5

K = 128
NCHUNK = E // K
NW = 32
ROWS_PER_TILE = N // 16

F32 = jnp.float32


def _silu(x):
    return x * jax.nn.sigmoid(x)


def _ln(x, g, b):
    mu = jnp.mean(x, axis=-1, keepdims=True)
    var = jnp.mean((x - mu) * (x - mu), axis=-1, keepdims=True)
    return (x - mu) * jax.lax.rsqrt(var + EPS) * g + b


_BN = 1000
_BE = 4000


def _node_enc_body(x_ref, wne_ref, bne_ref, gne_ref, bene_ref,
                   w1_ref, bm_ref, w2_ref, h_ref, a_ref, b_ref):
    t = jnp.dot(x_ref[...], wne_ref[...], preferred_element_type=F32)
    h = _silu(_ln(t + bne_ref[...], gne_ref[...], bene_ref[...]))
    h_ref[...] = h
    a_ref[...] = jnp.dot(h, w1_ref[...], preferred_element_type=F32) + bm_ref[...]
    b_ref[...] = jnp.dot(h, w2_ref[...], preferred_element_type=F32)


def _node_enc(x, wne, bne, gne, bene, w1, bm, w2):
    grid = (N // _BN,)
    row = pl.BlockSpec((_BN, None), lambda i: (i, 0))
    full = pl.BlockSpec((None, None), lambda i: (0, 0))

    def fixed(shape):
        return pl.BlockSpec(shape, lambda i: (0, 0))

    return pl.pallas_call(
        _node_enc_body,
        grid=grid,
        in_specs=[pl.BlockSpec((_BN, DIN), lambda i: (i, 0)),
                  fixed((DIN, H)), fixed((1, H)), fixed((1, H)), fixed((1, H)),
                  fixed((H, H)), fixed((1, H)), fixed((H, H))],
        out_specs=[pl.BlockSpec((_BN, H), lambda i: (i, 0))] * 3,
        out_shape=[jax.ShapeDtypeStruct((N, H), F32)] * 3,
    )(x, wne, bne, gne, bene, w1, bm, w2)


def _edge_enc_body(ea_ref, wee_ref, bee_ref, gee_ref, beee_ref,
                   w30_ref, w31_ref, c0_ref, c1_ref):
    t = jnp.dot(ea_ref[...], wee_ref[...], preferred_element_type=F32)
    e = _silu(_ln(t + bee_ref[...], gee_ref[...], beee_ref[...]))
    c0_ref[...] = jnp.dot(e, w30_ref[...], preferred_element_type=F32)
    c1_ref[...] = jnp.dot(e, w31_ref[...], preferred_element_type=F32)


def _edge_enc(ea, wee, bee, gee, beee, w30, w31):
    grid = (E // _BE,)

    def fixed(shape):
        return pl.BlockSpec(shape, lambda i: (0, 0))

    return pl.pallas_call(
        _edge_enc_body,
        grid=grid,
        in_specs=[pl.BlockSpec((_BE, DE), lambda i: (i, 0)),
                  fixed((DE, H)), fixed((1, H)), fixed((1, H)), fixed((1, H)),
                  fixed((H, H)), fixed((H, H))],
        out_specs=[pl.BlockSpec((_BE, H), lambda i: (i, 0))] * 2,
        out_shape=[jax.ShapeDtypeStruct((E, H), F32)] * 2,
    )(ea, wee, bee, gee, beee, w30, w31)


def _update_body_ab(h_ref, a0_ref, a1_ref, wu_ref, bu_ref, gu_ref, beu_ref,
                    gn_ref, ben_ref, w1_ref, bm_ref, w2_ref,
                    hn_ref, an_ref, bn_ref):
    h = h_ref[...]
    aggr = jnp.concatenate([a0_ref[...], a1_ref[...]], axis=-1)
    t = jnp.dot(h + aggr, wu_ref[...], preferred_element_type=F32) + bu_ref[...]
    u = _ln(_silu(t), gu_ref[...], beu_ref[...])
    hn = _ln(u + h, gn_ref[...], ben_ref[...])
    hn_ref[...] = hn
    an_ref[...] = jnp.dot(hn, w1_ref[...], preferred_element_type=F32) + bm_ref[...]
    bn_ref[...] = jnp.dot(hn, w2_ref[...], preferred_element_type=F32)


def _update_body(h_ref, a0_ref, a1_ref, wu_ref, bu_ref, gu_ref, beu_ref,
                 gn_ref, ben_ref, hn_ref):
    h = h_ref[...]
    aggr = jnp.concatenate([a0_ref[...], a1_ref[...]], axis=-1)
    t = jnp.dot(h + aggr, wu_ref[...], preferred_element_type=F32) + bu_ref[...]
    u = _ln(_silu(t), gu_ref[...], beu_ref[...])
    hn_ref[...] = _ln(u + h, gn_ref[...], ben_ref[...])


def _update(h, ag0, ag1, wu, bu, gu, beu, gn, ben, w1=None, bm=None, w2=None):
    grid = (N // _BN,)

    def fixed(shape):
        return pl.BlockSpec(shape, lambda i: (0, 0))

    row64 = pl.BlockSpec((_BN, H), lambda i: (i, 0))
    row32 = pl.BlockSpec((_BN, H // 2), lambda i: (i, 0))
    if w1 is not None:
        return pl.pallas_call(
            _update_body_ab,
            grid=grid,
            in_specs=[row64, row32, row32,
                      fixed((H, H)), fixed((1, H)), fixed((1, H)), fixed((1, H)),
                      fixed((1, H)), fixed((1, H)),
                      fixed((H, H)), fixed((1, H)), fixed((H, H))],
            out_specs=[row64] * 3,
            out_shape=[jax.ShapeDtypeStruct((N, H), F32)] * 3,
        )(h, ag0, ag1, wu, bu, gu, beu, gn, ben, w1, bm, w2)
    return pl.pallas_call(
        _update_body,
        grid=grid,
        in_specs=[row64, row32, row32,
                  fixed((H, H)), fixed((1, H)), fixed((1, H)), fixed((1, H)),
                  fixed((1, H)), fixed((1, H))],
        out_specs=row64,
        out_shape=jax.ShapeDtypeStruct((N, H), F32),
    )(h, ag0, ag1, wu, bu, gu, beu, gn, ben)


def _pool_head_body(h_ref, batch_ref, wh1_ref, bh1_ref, gh_ref, beh_ref,
                    wh2_ref, bh2_ref, out_ref, sums_ref, cnt_ref):
    i = pl.program_id(0)

    @pl.when(i == 0)
    def _():
        sums_ref[...] = jnp.zeros_like(sums_ref)
        cnt_ref[...] = jnp.zeros_like(cnt_ref)

    onehot = (batch_ref[...] == jax.lax.broadcasted_iota(jnp.int32, (1, G), 1)
              ).astype(F32)
    sums_ref[...] += jax.lax.dot_general(
        onehot, h_ref[...], (((0,), (0,)), ((), ())),
        preferred_element_type=F32)
    cnt_ref[...] += jax.lax.dot_general(
        onehot, jnp.ones((_BN, 1), F32), (((0,), (0,)), ((), ())),
        preferred_element_type=F32)

    @pl.when(i == pl.num_programs(0) - 1)
    def _():
        g = sums_ref[...] / jnp.maximum(cnt_ref[...], 1.0)
        z = _silu(_ln(jnp.dot(g, wh1_ref[...], preferred_element_type=F32)
                      + bh1_ref[...], gh_ref[...], beh_ref[...]))
        out_ref[...] = jnp.dot(z, wh2_ref[...], preferred_element_type=F32) \
            + bh2_ref[...]


def _pool_head(h, batch2d, wh1, bh1, gh, beh, wh2, bh2):
    grid = (N // _BN,)

    def fixed(shape):
        return pl.BlockSpec(shape, lambda i: (0, 0))

    return pl.pallas_call(
        _pool_head_body,
        grid=grid,
        in_specs=[pl.BlockSpec((_BN, H), lambda i: (i, 0)),
                  pl.BlockSpec((_BN, 1), lambda i: (i, 0)),
                  fixed((H, H // 2)), fixed((1, H // 2)), fixed((1, H // 2)),
                  fixed((1, H // 2)), fixed((H // 2, 1)), fixed((1, 1))],
        out_specs=fixed((G, 1)),
        out_shape=jax.ShapeDtypeStruct((G, 1), F32),
        scratch_shapes=[pltpu.VMEM((G, H), F32), pltpu.VMEM((G, 1), F32)],
    )(h, batch2d, wh1, bh1, gh, beh, wh2, bh2)


_WPER = NCHUNK // NW
_WREM = NCHUNK - _WPER * NW
_TPER = NCHUNK // 16
_TREM = NCHUNK - _TPER * 16


def _allsum16(v):
    for stride in (8, 4, 2, 1):
        idx = lax.iota(jnp.int32, 16) ^ stride
        v = v + v.at[idx].get(mode="promise_in_bounds", unique_indices=True)
    return v


def _edge_math(av, bv, cv, gvecs, bevecs, i, m0v, m1v):
    s = []
    for j in range(4):
        pre = (av[i, pl.ds(16 * j, 16)] + bv[i, pl.ds(16 * j, 16)]
               + cv[i, pl.ds(16 * j, 16)])
        s.append(pre / (1.0 + jnp.exp(-pre)))
    tot = (s[0] + s[1]) + (s[2] + s[3])
    mu = _allsum16(tot) * (1.0 / 64.0)
    d = [sj - mu for sj in s]
    sq = (d[0] * d[0] + d[1] * d[1]) + (d[2] * d[2] + d[3] * d[3])
    var = _allsum16(sq) * (1.0 / 64.0)
    vv = var + EPS
    bits = lax.bitcast_convert_type(vv, jnp.int32)
    y = lax.bitcast_convert_type(jnp.int32(0x5F3759DF) - (bits >> 1), F32)
    for _ in range(3):
        y = y * (1.5 - 0.5 * vv * y * y)
    for j in range(4):
        mj = d[j] * y * gvecs[j] + bevecs[j]
        if j < 2:
            m0v[i, pl.ds(16 * j, 16)] = mj
        else:
            m1v[i, pl.ds(16 * (j - 2), 16)] = mj


@functools.cache
def _make_msg_kernel():
    mesh = plsc.VectorSubcoreMesh(core_axis_name="c", subcore_axis_name="s")
    return functools.partial(
        pl.kernel,
        mesh=mesh,
        compiler_params=pltpu.CompilerParams(use_tc_tiling_on_sc=False),
        out_type=[jax.ShapeDtypeStruct((E, H // 2), F32),
                  jax.ShapeDtypeStruct((E, H // 2), F32)],
        scratch_types=[
            pltpu.VMEM((K,), jnp.int32),
            pltpu.VMEM((K,), jnp.int32),
            pltpu.VMEM((K, H), F32),
            pltpu.VMEM((K, H), F32),
            pltpu.VMEM((K, H), F32),
            pltpu.VMEM((K, H // 2), F32),
            pltpu.VMEM((K, H // 2), F32),
            pltpu.VMEM((H,), F32),
            pltpu.VMEM((H,), F32),
            pltpu.SemaphoreType.DMA,
        ],
    )(_msg_body)


def _msg_body(a_hbm, b_hbm, c_hbm, dst_hbm, src_hbm, gm_hbm, bem_hbm,
              m0_hbm, m1_hbm,
              dstv, srcv, av, bv, cv, m0v, m1v, gv, bev, sem):
    cid = lax.axis_index("c")
    sid = lax.axis_index("s")
    w = sid * 2 + cid
    nw = _WPER + jnp.where(w < _WREM, 1, 0)
    base = w * _WPER + jnp.minimum(w, _WREM)

    pltpu.sync_copy(gm_hbm, gv)
    pltpu.sync_copy(bem_hbm, bev)
    gvecs = [gv[pl.ds(16 * j, 16)] for j in range(4)]
    bevecs = [bev[pl.ds(16 * j, 16)] for j in range(4)]

    def chunk_body(j, carry):
        off = (base + j) * K
        pltpu.sync_copy(dst_hbm.at[pl.ds(off, K)], dstv)
        pltpu.sync_copy(src_hbm.at[pl.ds(off, K)], srcv)
        cp_a = pltpu.async_copy(a_hbm.at[dstv], av, sem)
        cp_b = pltpu.async_copy(b_hbm.at[srcv], bv, sem)
        cp_c = pltpu.async_copy(c_hbm.at[pl.ds(off, K)], cv, sem)
        cp_a.wait()
        cp_b.wait()
        cp_c.wait()

        def edge_body(i, carry2):
            _edge_math(av, bv, cv, gvecs, bevecs, i, m0v, m1v)
            return carry2

        lax.fori_loop(0, K, edge_body, 0)
        pltpu.sync_copy(m0v, m0_hbm.at[pl.ds(off, K)])
        pltpu.sync_copy(m1v, m1_hbm.at[pl.ds(off, K)])
        return carry

    lax.fori_loop(0, nw, chunk_body, 0)


@functools.cache
def _make_scatter_kernel():
    mesh = plsc.VectorSubcoreMesh(core_axis_name="c", subcore_axis_name="s")
    return functools.partial(
        pl.kernel,
        mesh=mesh,
        compiler_params=pltpu.CompilerParams(use_tc_tiling_on_sc=False),
        out_type=[jax.ShapeDtypeStruct((N, H // 2), F32),
                  jax.ShapeDtypeStruct((N, H // 2), F32)],
        scratch_types=[
            pltpu.VMEM((K,), jnp.int32),
            pltpu.VMEM((K, H // 2), F32),
            pltpu.VMEM_SHARED((N, H // 2), F32),
        ],
    )(_scatter_body)


def _scatter_body(m0_hbm, m1_hbm, dst_hbm, zeros_hbm, out0_hbm, out1_hbm,
                  dstv, mv, acc):
    cid = lax.axis_index("c")
    sid = lax.axis_index("s")
    nt = _TPER + jnp.where(sid < _TREM, 1, 0)
    base = sid * _TPER + jnp.minimum(sid, _TREM)
    row0 = sid * ROWS_PER_TILE

    pltpu.sync_copy(zeros_hbm.at[pl.ds(row0, ROWS_PER_TILE)],
                    acc.at[pl.ds(row0, ROWS_PER_TILE)])
    plsc.subcore_barrier()

    def chunk_body(j, carry):
        off = (base + j) * K
        pltpu.sync_copy(dst_hbm.at[pl.ds(off, K)], dstv)

        @pl.when(cid == 0)
        def _():
            pltpu.sync_copy(m0_hbm.at[pl.ds(off, K)], mv)

        @pl.when(cid == 1)
        def _():
            pltpu.sync_copy(m1_hbm.at[pl.ds(off, K)], mv)

        pltpu.sync_copy(mv, acc.at[dstv], add=True)
        return carry

    lax.fori_loop(0, nt, chunk_body, 0)
    plsc.subcore_barrier()

    @pl.when(cid == 0)
    def _():
        pltpu.sync_copy(acc.at[pl.ds(row0, ROWS_PER_TILE)],
                        out0_hbm.at[pl.ds(row0, ROWS_PER_TILE)])

    @pl.when(cid == 1)
    def _():
        pltpu.sync_copy(acc.at[pl.ds(row0, ROWS_PER_TILE)],
                        out1_hbm.at[pl.ds(row0, ROWS_PER_TILE)])


def kernel(x, edge_index, edge_attr, batch, params):
    p = params
    src = edge_index[0].astype(jnp.int32)
    dst = edge_index[1].astype(jnp.int32)
    batch2d = batch.astype(jnp.int32).reshape(N, 1)

    def r2(v):
        return v.reshape(1, -1)

    wm = [p['W_msg%d' % l] for l in range(2)]
    w1 = [w[0:H] for w in wm]
    w2 = [w[H:2 * H] for w in wm]
    w3 = [w[2 * H:3 * H] for w in wm]

    h, a_proj, b_proj = _node_enc(
        x, p['W_ne'], r2(p['b_ne']), r2(p['g_ne']), r2(p['be_ne']),
        w1[0], r2(p['b_msg0']), w2[0])
    c0, c1 = _edge_enc(
        edge_attr, p['W_ee'], r2(p['b_ee']), r2(p['g_ee']), r2(p['be_ee']),
        w3[0], w3[1])
    cs = [c0, c1]
    zeros = jnp.zeros((N, H // 2), F32)

    msg_fn = _make_msg_kernel()
    scatter_fn = _make_scatter_kernel()
    for l in range(2):
        m0, m1 = msg_fn(a_proj, b_proj, cs[l], dst, src,
                        p['g_msg%d' % l], p['be_msg%d' % l])
        ag0, ag1 = scatter_fn(m0, m1, dst, zeros)
        if l == 0:
            h, a_proj, b_proj = _update(
                h, ag0, ag1, p['W_upd0'], r2(p['b_upd0']), r2(p['g_upd0']),
                r2(p['be_upd0']), r2(p['g_n0']), r2(p['be_n0']),
                w1[1], r2(p['b_msg1']), w2[1])
        else:
            h = _update(
                h, ag0, ag1, p['W_upd1'], r2(p['b_upd1']), r2(p['g_upd1']),
                r2(p['be_upd1']), r2(p['g_n1']), r2(p['be_n1']))

    return _pool_head(h, batch2d, p['W_h1'], r2(p['b_h1']), r2(p['g_h']),
                      r2(p['be_h']), p['W_h2'], r2(p['b_h2']))


# speedup vs baseline: 1.7026x; 1.1292x over previous
import functools

import jax
import jax.numpy as jnp
from jax import lax
from jax.experimental import pallas as pl
from jax.experimental.pallas import tpu as pltpu
from jax.experimental.pallas import tpu_sc as plsc

N = 50000
E = 800000
DIN = 128
DE = 16
H = 64
G = 64
EPS = 1e-5

K = 128
NCHUNK = E // K
NW = 32
ROWS_PER_TILE = N // 16

F32 = jnp.float32


def _silu(x):
    return x * jax.nn.sigmoid(x)


def _ln(x, g, b):
    mu = jnp.mean(x, axis=-1, keepdims=True)
    var = jnp.mean((x - mu) * (x - mu), axis=-1, keepdims=True)
    return (x - mu) * jax.lax.rsqrt(var + EPS) * g + b


_BN = 1000
_BE = 4000


def _node_enc_body(x_ref, wne_ref, bne_ref, gne_ref, bene_ref,
                   w1_ref, bm_ref, w2_ref, h_ref, a_ref, b_ref):
    t = jnp.dot(x_ref[...], wne_ref[...], preferred_element_type=F32)
    h = _silu(_ln(t + bne_ref[...], gne_ref[...], bene_ref[...]))
    h_ref[...] = h
    a_ref[...] = jnp.dot(h, w1_ref[...], preferred_element_type=F32) + bm_ref[...]
    b_ref[...] = jnp.dot(h, w2_ref[...], preferred_element_type=F32)


def _node_enc(x, wne, bne, gne, bene, w1, bm, w2):
    grid = (N // _BN,)
    row = pl.BlockSpec((_BN, None), lambda i: (i, 0))
    full = pl.BlockSpec((None, None), lambda i: (0, 0))

    def fixed(shape):
        return pl.BlockSpec(shape, lambda i: (0, 0))

    return pl.pallas_call(
        _node_enc_body,
        grid=grid,
        in_specs=[pl.BlockSpec((_BN, DIN), lambda i: (i, 0)),
                  fixed((DIN, H)), fixed((1, H)), fixed((1, H)), fixed((1, H)),
                  fixed((H, H)), fixed((1, H)), fixed((H, H))],
        out_specs=[pl.BlockSpec((_BN, H), lambda i: (i, 0))] * 3,
        out_shape=[jax.ShapeDtypeStruct((N, H), F32)] * 3,
    )(x, wne, bne, gne, bene, w1, bm, w2)


def _edge_enc_body(ea_ref, wee_ref, bee_ref, gee_ref, beee_ref,
                   w30_ref, w31_ref, c0_ref, c1_ref):
    t = jnp.dot(ea_ref[...], wee_ref[...], preferred_element_type=F32)
    e = _silu(_ln(t + bee_ref[...], gee_ref[...], beee_ref[...]))
    c0_ref[...] = jnp.dot(e, w30_ref[...], preferred_element_type=F32)
    c1_ref[...] = jnp.dot(e, w31_ref[...], preferred_element_type=F32)


def _edge_enc(ea, wee, bee, gee, beee, w30, w31):
    grid = (E // _BE,)

    def fixed(shape):
        return pl.BlockSpec(shape, lambda i: (0, 0))

    return pl.pallas_call(
        _edge_enc_body,
        grid=grid,
        in_specs=[pl.BlockSpec((_BE, DE), lambda i: (i, 0)),
                  fixed((DE, H)), fixed((1, H)), fixed((1, H)), fixed((1, H)),
                  fixed((H, H)), fixed((H, H))],
        out_specs=[pl.BlockSpec((_BE, H), lambda i: (i, 0))] * 2,
        out_shape=[jax.ShapeDtypeStruct((E, H), F32)] * 2,
    )(ea, wee, bee, gee, beee, w30, w31)


def _update_body_ab(h_ref, a0_ref, a1_ref, wu_ref, bu_ref, gu_ref, beu_ref,
                    gn_ref, ben_ref, w1_ref, bm_ref, w2_ref,
                    hn_ref, an_ref, bn_ref):
    h = h_ref[...]
    aggr = jnp.concatenate([a0_ref[...], a1_ref[...]], axis=-1)
    t = jnp.dot(h + aggr, wu_ref[...], preferred_element_type=F32) + bu_ref[...]
    u = _ln(_silu(t), gu_ref[...], beu_ref[...])
    hn = _ln(u + h, gn_ref[...], ben_ref[...])
    hn_ref[...] = hn
    an_ref[...] = jnp.dot(hn, w1_ref[...], preferred_element_type=F32) + bm_ref[...]
    bn_ref[...] = jnp.dot(hn, w2_ref[...], preferred_element_type=F32)


def _update_body(h_ref, a0_ref, a1_ref, wu_ref, bu_ref, gu_ref, beu_ref,
                 gn_ref, ben_ref, hn_ref):
    h = h_ref[...]
    aggr = jnp.concatenate([a0_ref[...], a1_ref[...]], axis=-1)
    t = jnp.dot(h + aggr, wu_ref[...], preferred_element_type=F32) + bu_ref[...]
    u = _ln(_silu(t), gu_ref[...], beu_ref[...])
    hn_ref[...] = _ln(u + h, gn_ref[...], ben_ref[...])


def _update(h, ag0, ag1, wu, bu, gu, beu, gn, ben, w1=None, bm=None, w2=None):
    grid = (N // _BN,)

    def fixed(shape):
        return pl.BlockSpec(shape, lambda i: (0, 0))

    row64 = pl.BlockSpec((_BN, H), lambda i: (i, 0))
    row32 = pl.BlockSpec((_BN, H // 2), lambda i: (i, 0))
    if w1 is not None:
        return pl.pallas_call(
            _update_body_ab,
            grid=grid,
            in_specs=[row64, row32, row32,
                      fixed((H, H)), fixed((1, H)), fixed((1, H)), fixed((1, H)),
                      fixed((1, H)), fixed((1, H)),
                      fixed((H, H)), fixed((1, H)), fixed((H, H))],
            out_specs=[row64] * 3,
            out_shape=[jax.ShapeDtypeStruct((N, H), F32)] * 3,
        )(h, ag0, ag1, wu, bu, gu, beu, gn, ben, w1, bm, w2)
    return pl.pallas_call(
        _update_body,
        grid=grid,
        in_specs=[row64, row32, row32,
                  fixed((H, H)), fixed((1, H)), fixed((1, H)), fixed((1, H)),
                  fixed((1, H)), fixed((1, H))],
        out_specs=row64,
        out_shape=jax.ShapeDtypeStruct((N, H), F32),
    )(h, ag0, ag1, wu, bu, gu, beu, gn, ben)


def _pool_head_body(h_ref, batch_ref, wh1_ref, bh1_ref, gh_ref, beh_ref,
                    wh2_ref, bh2_ref, out_ref, sums_ref, cnt_ref):
    i = pl.program_id(0)

    @pl.when(i == 0)
    def _():
        sums_ref[...] = jnp.zeros_like(sums_ref)
        cnt_ref[...] = jnp.zeros_like(cnt_ref)

    onehot = (batch_ref[...] == jax.lax.broadcasted_iota(jnp.int32, (1, G), 1)
              ).astype(F32)
    sums_ref[...] += jax.lax.dot_general(
        onehot, h_ref[...], (((0,), (0,)), ((), ())),
        preferred_element_type=F32)
    cnt_ref[...] += jax.lax.dot_general(
        onehot, jnp.ones((_BN, 1), F32), (((0,), (0,)), ((), ())),
        preferred_element_type=F32)

    @pl.when(i == pl.num_programs(0) - 1)
    def _():
        g = sums_ref[...] / jnp.maximum(cnt_ref[...], 1.0)
        z = _silu(_ln(jnp.dot(g, wh1_ref[...], preferred_element_type=F32)
                      + bh1_ref[...], gh_ref[...], beh_ref[...]))
        out_ref[...] = jnp.dot(z, wh2_ref[...], preferred_element_type=F32) \
            + bh2_ref[...]


def _pool_head(h, batch2d, wh1, bh1, gh, beh, wh2, bh2):
    grid = (N // _BN,)

    def fixed(shape):
        return pl.BlockSpec(shape, lambda i: (0, 0))

    return pl.pallas_call(
        _pool_head_body,
        grid=grid,
        in_specs=[pl.BlockSpec((_BN, H), lambda i: (i, 0)),
                  pl.BlockSpec((_BN, 1), lambda i: (i, 0)),
                  fixed((H, H // 2)), fixed((1, H // 2)), fixed((1, H // 2)),
                  fixed((1, H // 2)), fixed((H // 2, 1)), fixed((1, 1))],
        out_specs=fixed((G, 1)),
        out_shape=jax.ShapeDtypeStruct((G, 1), F32),
        scratch_shapes=[pltpu.VMEM((G, H), F32), pltpu.VMEM((G, 1), F32)],
    )(h, batch2d, wh1, bh1, gh, beh, wh2, bh2)


_WPER = NCHUNK // NW
_WREM = NCHUNK - _WPER * NW
_TPER = NCHUNK // 16
_TREM = NCHUNK - _TPER * 16


def _allsum16(v):
    for stride in (8, 4, 2, 1):
        idx = lax.iota(jnp.int32, 16) ^ stride
        v = v + v.at[idx].get(mode="promise_in_bounds", unique_indices=True)
    return v


def _edge_math(av, bv, cv, gvecs, bevecs, i, m0v, m1v):
    s = []
    for j in range(4):
        pre = (av[i, pl.ds(16 * j, 16)] + bv[i, pl.ds(16 * j, 16)]
               + cv[i, pl.ds(16 * j, 16)])
        s.append(pre / (1.0 + jnp.exp(-pre)))
    tot = (s[0] + s[1]) + (s[2] + s[3])
    mu = _allsum16(tot) * (1.0 / 64.0)
    d = [sj - mu for sj in s]
    sq = (d[0] * d[0] + d[1] * d[1]) + (d[2] * d[2] + d[3] * d[3])
    var = _allsum16(sq) * (1.0 / 64.0)
    vv = var + EPS
    bits = lax.bitcast_convert_type(vv, jnp.int32)
    y = lax.bitcast_convert_type(jnp.int32(0x5F3759DF) - (bits >> 1), F32)
    for _ in range(3):
        y = y * (1.5 - 0.5 * vv * y * y)
    for j in range(4):
        mj = d[j] * y * gvecs[j] + bevecs[j]
        if j < 2:
            m0v[i, pl.ds(16 * j, 16)] = mj
        else:
            m1v[i, pl.ds(16 * (j - 2), 16)] = mj


_WSLOTS = _WPER + 1


@functools.cache
def _make_msg_kernel():
    mesh = plsc.VectorSubcoreMesh(core_axis_name="c", subcore_axis_name="s")
    return functools.partial(
        pl.kernel,
        mesh=mesh,
        compiler_params=pltpu.CompilerParams(use_tc_tiling_on_sc=False),
        out_type=[jax.ShapeDtypeStruct((E, H // 2), F32),
                  jax.ShapeDtypeStruct((E, H // 2), F32)],
        scratch_types=[
            [pltpu.VMEM((K,), jnp.int32)] * 2,
            [pltpu.VMEM((K,), jnp.int32)] * 2,
            [pltpu.VMEM((K, H), F32)] * 2,
            [pltpu.VMEM((K, H), F32)] * 2,
            [pltpu.VMEM((K, H), F32)] * 2,
            [pltpu.VMEM((K, H // 2), F32)] * 2,
            [pltpu.VMEM((K, H // 2), F32)] * 2,
            pltpu.VMEM((H,), F32),
            pltpu.VMEM((H,), F32),
            [pltpu.SemaphoreType.DMA] * 2,
            [pltpu.SemaphoreType.DMA] * 2,
            [pltpu.SemaphoreType.DMA] * 2,
        ],
    )(_msg_body)


def _msg_body(a_hbm, b_hbm, c_hbm, dst_hbm, src_hbm, gm_hbm, bem_hbm,
              m0_hbm, m1_hbm,
              dstv, srcv, av, bv, cv, m0v, m1v, gv, bev, sidx, srows, swb):
    cid = lax.axis_index("c")
    sid = lax.axis_index("s")
    w = sid * 2 + cid
    base = w * _WPER + jnp.minimum(w, _WREM)

    pltpu.sync_copy(gm_hbm, gv)
    pltpu.sync_copy(bem_hbm, bev)
    gvecs = [gv[pl.ds(16 * j, 16)] for j in range(4)]
    bevecs = [bev[pl.ds(16 * j, 16)] for j in range(4)]

    def off_of(j):
        return lax.rem(base + j, NCHUNK) * K

    def issue_idx(j, b):
        off = off_of(j)
        pltpu.async_copy(dst_hbm.at[pl.ds(off, K)], dstv[b], sidx[b])
        pltpu.async_copy(src_hbm.at[pl.ds(off, K)], srcv[b], sidx[b])

    def wait_idx(b):
        pltpu.make_async_copy(dst_hbm.at[pl.ds(0, K)], dstv[b], sidx[b]).wait()
        pltpu.make_async_copy(src_hbm.at[pl.ds(0, K)], srcv[b], sidx[b]).wait()

    def issue_rows(j, b):
        off = off_of(j)
        pltpu.async_copy(a_hbm.at[dstv[b]], av[b], srows[b])
        pltpu.async_copy(b_hbm.at[srcv[b]], bv[b], srows[b])
        pltpu.async_copy(c_hbm.at[pl.ds(off, K)], cv[b], srows[b])

    def wait_rows(b):
        pltpu.make_async_copy(c_hbm.at[pl.ds(0, K)], av[b], srows[b]).wait()
        pltpu.make_async_copy(c_hbm.at[pl.ds(0, K)], bv[b], srows[b]).wait()
        pltpu.make_async_copy(c_hbm.at[pl.ds(0, K)], cv[b], srows[b]).wait()

    def issue_wb(j, b):
        off = off_of(j)
        pltpu.async_copy(m0v[b], m0_hbm.at[pl.ds(off, K)], swb[b])
        pltpu.async_copy(m1v[b], m1_hbm.at[pl.ds(off, K)], swb[b])

    def wait_wb(b):
        pltpu.make_async_copy(m0_hbm.at[pl.ds(0, K)], m0v[b], swb[b]).wait()
        pltpu.make_async_copy(m1_hbm.at[pl.ds(0, K)], m1v[b], swb[b]).wait()

    def compute(b):
        def edge_body(q, carry2):
            for e in range(4):
                _edge_math(av[b], bv[b], cv[b], gvecs, bevecs, 4 * q + e,
                           m0v[b], m1v[b])
            return carry2

        lax.fori_loop(0, K // 4, edge_body, 0)

    issue_idx(0, 0)
    wait_idx(0)
    issue_rows(0, 0)
    issue_idx(1, 1)

    def pair_body(t, carry):
        for b in (0, 1):
            j = 2 * t + b
            wait_rows(b)
            issue_idx(j + 2, b)
            wait_idx(1 - b)
            issue_rows(j + 1, 1 - b)

            @pl.when(t > 0)
            def _():
                wait_wb(b)

            compute(b)
            issue_wb(j, b)
        return carry

    lax.fori_loop(0, _WSLOTS // 2, pair_body, 0)
    wait_rows(0)
    wait_idx(1)
    wait_wb(0)
    wait_wb(1)


_EB = 80
_SC_CHUNKS = (E // 16) // _EB


@functools.cache
def _make_scatter_kernel():
    mesh = plsc.VectorSubcoreMesh(core_axis_name="c", subcore_axis_name="s")
    return functools.partial(
        pl.kernel,
        mesh=mesh,
        compiler_params=pltpu.CompilerParams(use_tc_tiling_on_sc=False),
        out_type=[jax.ShapeDtypeStruct((N, H // 2), F32),
                  jax.ShapeDtypeStruct((N, H // 2), F32)],
        scratch_types=[
            [pltpu.VMEM((_EB,), jnp.int32)] * 2,
            [pltpu.VMEM((_EB, H // 2), F32)] * 2,
            pltpu.VMEM_SHARED((N, H // 2), F32),
            [pltpu.SemaphoreType.DMA] * 2,
            pltpu.SemaphoreType.DMA,
        ],
    )(_scatter_body)


def _scatter_body(m0_hbm, m1_hbm, dst_hbm, zeros_hbm, out0_hbm, out1_hbm,
                  dstv, mv, acc, sload, ssc):
    cid = lax.axis_index("c")
    sid = lax.axis_index("s")
    ebase = sid * (E // 16)
    row0 = sid * ROWS_PER_TILE

    pltpu.sync_copy(zeros_hbm.at[pl.ds(row0, ROWS_PER_TILE)],
                    acc.at[pl.ds(row0, ROWS_PER_TILE)])
    plsc.subcore_barrier()

    def issue_loads(j, b):
        off = ebase + j * _EB
        pltpu.async_copy(dst_hbm.at[pl.ds(off, _EB)], dstv[b], sload[b])

        @pl.when(cid == 0)
        def _():
            pltpu.async_copy(m0_hbm.at[pl.ds(off, _EB)], mv[b], sload[b])

        @pl.when(cid == 1)
        def _():
            pltpu.async_copy(m1_hbm.at[pl.ds(off, _EB)], mv[b], sload[b])

    def wait_loads(b):
        pltpu.make_async_copy(dst_hbm.at[pl.ds(0, _EB)], dstv[b],
                              sload[b]).wait()
        pltpu.make_async_copy(m0_hbm.at[pl.ds(0, _EB)], mv[b],
                              sload[b]).wait()

    def scatter(b):
        pltpu.async_copy(mv[b], acc.at[dstv[b]], ssc, add=True).wait()

    issue_loads(0, 0)
    issue_loads(1, 1)

    def pair_body(t, carry):
        for b in (0, 1):
            j = 2 * t + b
            wait_loads(b)
            scatter(b)

            @pl.when(j + 2 < _SC_CHUNKS)
            def _():
                issue_loads(j + 2, b)
        return carry

    lax.fori_loop(0, _SC_CHUNKS // 2, pair_body, 0)
    wait_loads(0)
    scatter(0)
    plsc.subcore_barrier()

    @pl.when(cid == 0)
    def _():
        pltpu.sync_copy(acc.at[pl.ds(row0, ROWS_PER_TILE)],
                        out0_hbm.at[pl.ds(row0, ROWS_PER_TILE)])

    @pl.when(cid == 1)
    def _():
        pltpu.sync_copy(acc.at[pl.ds(row0, ROWS_PER_TILE)],
                        out1_hbm.at[pl.ds(row0, ROWS_PER_TILE)])


def kernel(x, edge_index, edge_attr, batch, params):
    p = params
    src = edge_index[0].astype(jnp.int32)
    dst = edge_index[1].astype(jnp.int32)
    batch2d = batch.astype(jnp.int32).reshape(N, 1)

    def r2(v):
        return v.reshape(1, -1)

    wm = [p['W_msg%d' % l] for l in range(2)]
    w1 = [w[0:H] for w in wm]
    w2 = [w[H:2 * H] for w in wm]
    w3 = [w[2 * H:3 * H] for w in wm]

    h, a_proj, b_proj = _node_enc(
        x, p['W_ne'], r2(p['b_ne']), r2(p['g_ne']), r2(p['be_ne']),
        w1[0], r2(p['b_msg0']), w2[0])
    c0, c1 = _edge_enc(
        edge_attr, p['W_ee'], r2(p['b_ee']), r2(p['g_ee']), r2(p['be_ee']),
        w3[0], w3[1])
    cs = [c0, c1]
    zeros = jnp.zeros((N, H // 2), F32)

    msg_fn = _make_msg_kernel()
    scatter_fn = _make_scatter_kernel()
    for l in range(2):
        m0, m1 = msg_fn(a_proj, b_proj, cs[l], dst, src,
                        p['g_msg%d' % l], p['be_msg%d' % l])
        ag0, ag1 = scatter_fn(m0, m1, dst, zeros)
        if l == 0:
            h, a_proj, b_proj = _update(
                h, ag0, ag1, p['W_upd0'], r2(p['b_upd0']), r2(p['g_upd0']),
                r2(p['be_upd0']), r2(p['g_n0']), r2(p['be_n0']),
                w1[1], r2(p['b_msg1']), w2[1])
        else:
            h = _update(
                h, ag0, ag1, p['W_upd1'], r2(p['b_upd1']), r2(p['g_upd1']),
                r2(p['be_upd1']), r2(p['g_n1']), r2(p['be_n1']))

    return _pool_head(h, batch2d, p['W_h1'], r2(p['b_h1']), r2(p['g_h']),
                      r2(p['be_h']), p['W_h2'], r2(p['b_h2']))


# speedup vs baseline: 3.7118x; 2.1801x over previous
import functools

import jax
import jax.numpy as jnp
from jax import lax
from jax.experimental import pallas as pl
from jax.experimental.pallas import tpu as pltpu
from jax.experimental.pallas import tpu_sc as plsc

N = 50000
E = 800000
DIN = 128
DE = 16
H = 64
G = 64
EPS = 1e-5

K = 128
NCHUNK = E // K
NW = 32
ROWS_PER_TILE = N // 16

F32 = jnp.float32


def _silu(x):
    return x * jax.nn.sigmoid(x)


def _ln(x, g, b):
    mu = jnp.mean(x, axis=-1, keepdims=True)
    var = jnp.mean((x - mu) * (x - mu), axis=-1, keepdims=True)
    return (x - mu) * jax.lax.rsqrt(var + EPS) * g + b


_BN = 1000
_BE = 4000


def _node_enc_body(x_ref, wne_ref, bne_ref, gne_ref, bene_ref,
                   w1_ref, bm_ref, w2_ref, h_ref, a_ref, b_ref):
    t = jnp.dot(x_ref[...], wne_ref[...], preferred_element_type=F32)
    h = _silu(_ln(t + bne_ref[...], gne_ref[...], bene_ref[...]))
    h_ref[...] = h
    a_ref[...] = jnp.dot(h, w1_ref[...], preferred_element_type=F32) + bm_ref[...]
    b_ref[...] = jnp.dot(h, w2_ref[...], preferred_element_type=F32)


def _node_enc(x, wne, bne, gne, bene, w1, bm, w2):
    grid = (N // _BN,)
    row = pl.BlockSpec((_BN, None), lambda i: (i, 0))
    full = pl.BlockSpec((None, None), lambda i: (0, 0))

    def fixed(shape):
        return pl.BlockSpec(shape, lambda i: (0, 0))

    return pl.pallas_call(
        _node_enc_body,
        grid=grid,
        in_specs=[pl.BlockSpec((_BN, DIN), lambda i: (i, 0)),
                  fixed((DIN, H)), fixed((1, H)), fixed((1, H)), fixed((1, H)),
                  fixed((H, H)), fixed((1, H)), fixed((H, H))],
        out_specs=[pl.BlockSpec((_BN, H), lambda i: (i, 0))] * 3,
        out_shape=[jax.ShapeDtypeStruct((N, H), F32)] * 3,
    )(x, wne, bne, gne, bene, w1, bm, w2)


def _edge_enc_body(ea_ref, wee_ref, bee_ref, gee_ref, beee_ref,
                   w30_ref, w31_ref, c0_ref, c1_ref):
    t = jnp.dot(ea_ref[...], wee_ref[...], preferred_element_type=F32)
    e = _silu(_ln(t + bee_ref[...], gee_ref[...], beee_ref[...]))
    c0_ref[...] = jnp.dot(e, w30_ref[...], preferred_element_type=F32)
    c1_ref[...] = jnp.dot(e, w31_ref[...], preferred_element_type=F32)


def _edge_enc(ea, wee, bee, gee, beee, w30, w31):
    grid = (E // _BE,)

    def fixed(shape):
        return pl.BlockSpec(shape, lambda i: (0, 0))

    return pl.pallas_call(
        _edge_enc_body,
        grid=grid,
        in_specs=[pl.BlockSpec((_BE, DE), lambda i: (i, 0)),
                  fixed((DE, H)), fixed((1, H)), fixed((1, H)), fixed((1, H)),
                  fixed((H, H)), fixed((H, H))],
        out_specs=[pl.BlockSpec((_BE, H), lambda i: (i, 0))] * 2,
        out_shape=[jax.ShapeDtypeStruct((E, H), F32)] * 2,
    )(ea, wee, bee, gee, beee, w30, w31)


def _update_body_ab(h_ref, a0_ref, a1_ref, wu_ref, bu_ref, gu_ref, beu_ref,
                    gn_ref, ben_ref, w1_ref, bm_ref, w2_ref,
                    hn_ref, an_ref, bn_ref):
    h = h_ref[...]
    aggr = jnp.concatenate([a0_ref[...], a1_ref[...]], axis=-1)
    t = jnp.dot(h + aggr, wu_ref[...], preferred_element_type=F32) + bu_ref[...]
    u = _ln(_silu(t), gu_ref[...], beu_ref[...])
    hn = _ln(u + h, gn_ref[...], ben_ref[...])
    hn_ref[...] = hn
    an_ref[...] = jnp.dot(hn, w1_ref[...], preferred_element_type=F32) + bm_ref[...]
    bn_ref[...] = jnp.dot(hn, w2_ref[...], preferred_element_type=F32)


def _update_body(h_ref, a0_ref, a1_ref, wu_ref, bu_ref, gu_ref, beu_ref,
                 gn_ref, ben_ref, hn_ref):
    h = h_ref[...]
    aggr = jnp.concatenate([a0_ref[...], a1_ref[...]], axis=-1)
    t = jnp.dot(h + aggr, wu_ref[...], preferred_element_type=F32) + bu_ref[...]
    u = _ln(_silu(t), gu_ref[...], beu_ref[...])
    hn_ref[...] = _ln(u + h, gn_ref[...], ben_ref[...])


def _update(h, ag0, ag1, wu, bu, gu, beu, gn, ben, w1=None, bm=None, w2=None):
    grid = (N // _BN,)

    def fixed(shape):
        return pl.BlockSpec(shape, lambda i: (0, 0))

    row64 = pl.BlockSpec((_BN, H), lambda i: (i, 0))
    row32 = pl.BlockSpec((_BN, H // 2), lambda i: (i, 0))
    if w1 is not None:
        return pl.pallas_call(
            _update_body_ab,
            grid=grid,
            in_specs=[row64, row32, row32,
                      fixed((H, H)), fixed((1, H)), fixed((1, H)), fixed((1, H)),
                      fixed((1, H)), fixed((1, H)),
                      fixed((H, H)), fixed((1, H)), fixed((H, H))],
            out_specs=[row64] * 3,
            out_shape=[jax.ShapeDtypeStruct((N, H), F32)] * 3,
        )(h, ag0, ag1, wu, bu, gu, beu, gn, ben, w1, bm, w2)
    return pl.pallas_call(
        _update_body,
        grid=grid,
        in_specs=[row64, row32, row32,
                  fixed((H, H)), fixed((1, H)), fixed((1, H)), fixed((1, H)),
                  fixed((1, H)), fixed((1, H))],
        out_specs=row64,
        out_shape=jax.ShapeDtypeStruct((N, H), F32),
    )(h, ag0, ag1, wu, bu, gu, beu, gn, ben)


def _pool_head_body(h_ref, batch_ref, wh1_ref, bh1_ref, gh_ref, beh_ref,
                    wh2_ref, bh2_ref, out_ref, sums_ref, cnt_ref):
    i = pl.program_id(0)

    @pl.when(i == 0)
    def _():
        sums_ref[...] = jnp.zeros_like(sums_ref)
        cnt_ref[...] = jnp.zeros_like(cnt_ref)

    onehot = (batch_ref[...] == jax.lax.broadcasted_iota(jnp.int32, (1, G), 1)
              ).astype(F32)
    sums_ref[...] += jax.lax.dot_general(
        onehot, h_ref[...], (((0,), (0,)), ((), ())),
        preferred_element_type=F32)
    cnt_ref[...] += jax.lax.dot_general(
        onehot, jnp.ones((_BN, 1), F32), (((0,), (0,)), ((), ())),
        preferred_element_type=F32)

    @pl.when(i == pl.num_programs(0) - 1)
    def _():
        g = sums_ref[...] / jnp.maximum(cnt_ref[...], 1.0)
        z = _silu(_ln(jnp.dot(g, wh1_ref[...], preferred_element_type=F32)
                      + bh1_ref[...], gh_ref[...], beh_ref[...]))
        out_ref[...] = jnp.dot(z, wh2_ref[...], preferred_element_type=F32) \
            + bh2_ref[...]


def _pool_head(h, batch2d, wh1, bh1, gh, beh, wh2, bh2):
    grid = (N // _BN,)

    def fixed(shape):
        return pl.BlockSpec(shape, lambda i: (0, 0))

    return pl.pallas_call(
        _pool_head_body,
        grid=grid,
        in_specs=[pl.BlockSpec((_BN, H), lambda i: (i, 0)),
                  pl.BlockSpec((_BN, 1), lambda i: (i, 0)),
                  fixed((H, H // 2)), fixed((1, H // 2)), fixed((1, H // 2)),
                  fixed((1, H // 2)), fixed((H // 2, 1)), fixed((1, 1))],
        out_specs=fixed((G, 1)),
        out_shape=jax.ShapeDtypeStruct((G, 1), F32),
        scratch_shapes=[pltpu.VMEM((G, H), F32), pltpu.VMEM((G, 1), F32)],
    )(h, batch2d, wh1, bh1, gh, beh, wh2, bh2)


_WPER = NCHUNK // NW
_WREM = NCHUNK - _WPER * NW
_TPER = NCHUNK // 16
_TREM = NCHUNK - _TPER * 16


def _allsum16(v):
    for stride in (8, 4, 2, 1):
        idx = lax.iota(jnp.int32, 16) ^ stride
        v = v + v.at[idx].get(mode="promise_in_bounds", unique_indices=True)
    return v


def _edge_math(av, bv, cv, gvecs, bevecs, i, m0v, m1v):
    s = []
    for j in range(4):
        pre = (av[i, pl.ds(16 * j, 16)] + bv[i, pl.ds(16 * j, 16)]
               + cv[i, pl.ds(16 * j, 16)])
        s.append(pre / (1.0 + jnp.exp(-pre)))
    tot = (s[0] + s[1]) + (s[2] + s[3])
    mu = _allsum16(tot) * (1.0 / 64.0)
    d = [sj - mu for sj in s]
    sq = (d[0] * d[0] + d[1] * d[1]) + (d[2] * d[2] + d[3] * d[3])
    var = _allsum16(sq) * (1.0 / 64.0)
    vv = var + EPS
    bits = lax.bitcast_convert_type(vv, jnp.int32)
    y = lax.bitcast_convert_type(jnp.int32(0x5F3759DF) - (bits >> 1), F32)
    for _ in range(3):
        y = y * (1.5 - 0.5 * vv * y * y)
    for j in range(4):
        mj = d[j] * y * gvecs[j] + bevecs[j]
        if j < 2:
            m0v[i, pl.ds(16 * j, 16)] = mj
        else:
            m1v[i, pl.ds(16 * (j - 2), 16)] = mj


_WSLOTS = _WPER + 1


@functools.cache
def _make_msg_kernel():
    mesh = plsc.VectorSubcoreMesh(core_axis_name="c", subcore_axis_name="s")
    return functools.partial(
        pl.kernel,
        mesh=mesh,
        compiler_params=pltpu.CompilerParams(use_tc_tiling_on_sc=False),
        out_type=[jax.ShapeDtypeStruct((E, H // 2), F32),
                  jax.ShapeDtypeStruct((E, H // 2), F32)],
        scratch_types=[
            [pltpu.VMEM((K,), jnp.int32)] * 2,
            [pltpu.VMEM((K,), jnp.int32)] * 2,
            [pltpu.VMEM((K, H), F32)] * 2,
            [pltpu.VMEM((K, H), F32)] * 2,
            [pltpu.VMEM((K, H), F32)] * 2,
            [pltpu.VMEM((K, H // 2), F32)] * 2,
            [pltpu.VMEM((K, H // 2), F32)] * 2,
            pltpu.VMEM((H,), F32),
            pltpu.VMEM((H,), F32),
            [pltpu.SemaphoreType.DMA] * 2,
            [pltpu.SemaphoreType.DMA] * 2,
            [pltpu.SemaphoreType.DMA] * 2,
        ],
    )(_msg_body)


def _msg_body(a_hbm, b_hbm, c_hbm, dst_hbm, src_hbm, gm_hbm, bem_hbm,
              m0_hbm, m1_hbm,
              dstv, srcv, av, bv, cv, m0v, m1v, gv, bev, sidx, srows, swb):
    cid = lax.axis_index("c")
    sid = lax.axis_index("s")
    w = sid * 2 + cid
    base = w * _WPER + jnp.minimum(w, _WREM)

    pltpu.sync_copy(gm_hbm, gv)
    pltpu.sync_copy(bem_hbm, bev)
    gvecs = [gv[pl.ds(16 * j, 16)] for j in range(4)]
    bevecs = [bev[pl.ds(16 * j, 16)] for j in range(4)]

    def off_of(j):
        return lax.rem(base + j, NCHUNK) * K

    def issue_idx(j, b):
        off = off_of(j)
        pltpu.async_copy(dst_hbm.at[pl.ds(off, K)], dstv[b], sidx[b])
        pltpu.async_copy(src_hbm.at[pl.ds(off, K)], srcv[b], sidx[b])

    def wait_idx(b):
        pltpu.make_async_copy(dst_hbm.at[pl.ds(0, K)], dstv[b], sidx[b]).wait()
        pltpu.make_async_copy(src_hbm.at[pl.ds(0, K)], srcv[b], sidx[b]).wait()

    def issue_rows(j, b):
        off = off_of(j)
        pltpu.async_copy(a_hbm.at[dstv[b]], av[b], srows[b])
        pltpu.async_copy(b_hbm.at[srcv[b]], bv[b], srows[b])
        pltpu.async_copy(c_hbm.at[pl.ds(off, K)], cv[b], srows[b])

    def wait_rows(b):
        pltpu.make_async_copy(c_hbm.at[pl.ds(0, K)], av[b], srows[b]).wait()
        pltpu.make_async_copy(c_hbm.at[pl.ds(0, K)], bv[b], srows[b]).wait()
        pltpu.make_async_copy(c_hbm.at[pl.ds(0, K)], cv[b], srows[b]).wait()

    def issue_wb(j, b):
        off = off_of(j)
        pltpu.async_copy(m0v[b], m0_hbm.at[pl.ds(off, K)], swb[b])
        pltpu.async_copy(m1v[b], m1_hbm.at[pl.ds(off, K)], swb[b])

    def wait_wb(b):
        pltpu.make_async_copy(m0_hbm.at[pl.ds(0, K)], m0v[b], swb[b]).wait()
        pltpu.make_async_copy(m1_hbm.at[pl.ds(0, K)], m1v[b], swb[b]).wait()

    def compute(b):
        @plsc.parallel_loop(0, K, 1, unroll=4)
        def _edge(i):
            _edge_math(av[b], bv[b], cv[b], gvecs, bevecs, i, m0v[b], m1v[b])

    issue_idx(0, 0)
    wait_idx(0)
    issue_rows(0, 0)
    issue_idx(1, 1)

    def pair_body(t, carry):
        for b in (0, 1):
            j = 2 * t + b
            wait_rows(b)
            issue_idx(j + 2, b)
            wait_idx(1 - b)
            issue_rows(j + 1, 1 - b)

            @pl.when(t > 0)
            def _():
                wait_wb(b)

            compute(b)
            issue_wb(j, b)
        return carry

    lax.fori_loop(0, _WSLOTS // 2, pair_body, 0)
    wait_rows(0)
    wait_idx(1)
    wait_wb(0)
    wait_wb(1)


_EB = 80
_SC_CHUNKS = (E // 16) // _EB


@functools.cache
def _make_scatter_kernel():
    mesh = plsc.VectorSubcoreMesh(core_axis_name="c", subcore_axis_name="s")
    return functools.partial(
        pl.kernel,
        mesh=mesh,
        compiler_params=pltpu.CompilerParams(use_tc_tiling_on_sc=False),
        out_type=[jax.ShapeDtypeStruct((N, H // 2), F32),
                  jax.ShapeDtypeStruct((N, H // 2), F32)],
        scratch_types=[
            [pltpu.VMEM((_EB,), jnp.int32)] * 2,
            [pltpu.VMEM((_EB, H // 2), F32)] * 2,
            pltpu.VMEM_SHARED((N, H // 2), F32),
            [pltpu.SemaphoreType.DMA] * 2,
            pltpu.SemaphoreType.DMA,
        ],
    )(_scatter_body)


def _scatter_body(m0_hbm, m1_hbm, dst_hbm, zeros_hbm, out0_hbm, out1_hbm,
                  dstv, mv, acc, sload, ssc):
    cid = lax.axis_index("c")
    sid = lax.axis_index("s")
    ebase = sid * (E // 16)
    row0 = sid * ROWS_PER_TILE

    pltpu.sync_copy(zeros_hbm.at[pl.ds(row0, ROWS_PER_TILE)],
                    acc.at[pl.ds(row0, ROWS_PER_TILE)])
    plsc.subcore_barrier()

    def issue_loads(j, b):
        off = ebase + j * _EB
        pltpu.async_copy(dst_hbm.at[pl.ds(off, _EB)], dstv[b], sload[b])

        @pl.when(cid == 0)
        def _():
            pltpu.async_copy(m0_hbm.at[pl.ds(off, _EB)], mv[b], sload[b])

        @pl.when(cid == 1)
        def _():
            pltpu.async_copy(m1_hbm.at[pl.ds(off, _EB)], mv[b], sload[b])

    def wait_loads(b):
        pltpu.make_async_copy(dst_hbm.at[pl.ds(0, _EB)], dstv[b],
                              sload[b]).wait()
        pltpu.make_async_copy(m0_hbm.at[pl.ds(0, _EB)], mv[b],
                              sload[b]).wait()

    def scatter(b):
        pltpu.async_copy(mv[b], acc.at[dstv[b]], ssc, add=True).wait()

    issue_loads(0, 0)
    issue_loads(1, 1)

    def pair_body(t, carry):
        for b in (0, 1):
            j = 2 * t + b
            wait_loads(b)
            scatter(b)

            @pl.when(j + 2 < _SC_CHUNKS)
            def _():
                issue_loads(j + 2, b)
        return carry

    lax.fori_loop(0, _SC_CHUNKS // 2, pair_body, 0)
    wait_loads(0)
    scatter(0)
    plsc.subcore_barrier()

    @pl.when(cid == 0)
    def _():
        pltpu.sync_copy(acc.at[pl.ds(row0, ROWS_PER_TILE)],
                        out0_hbm.at[pl.ds(row0, ROWS_PER_TILE)])

    @pl.when(cid == 1)
    def _():
        pltpu.sync_copy(acc.at[pl.ds(row0, ROWS_PER_TILE)],
                        out1_hbm.at[pl.ds(row0, ROWS_PER_TILE)])


def kernel(x, edge_index, edge_attr, batch, params):
    p = params
    src = edge_index[0].astype(jnp.int32)
    dst = edge_index[1].astype(jnp.int32)
    batch2d = batch.astype(jnp.int32).reshape(N, 1)

    def r2(v):
        return v.reshape(1, -1)

    wm = [p['W_msg%d' % l] for l in range(2)]
    w1 = [w[0:H] for w in wm]
    w2 = [w[H:2 * H] for w in wm]
    w3 = [w[2 * H:3 * H] for w in wm]

    h, a_proj, b_proj = _node_enc(
        x, p['W_ne'], r2(p['b_ne']), r2(p['g_ne']), r2(p['be_ne']),
        w1[0], r2(p['b_msg0']), w2[0])
    c0, c1 = _edge_enc(
        edge_attr, p['W_ee'], r2(p['b_ee']), r2(p['g_ee']), r2(p['be_ee']),
        w3[0], w3[1])
    cs = [c0, c1]
    zeros = jnp.zeros((N, H // 2), F32)

    msg_fn = _make_msg_kernel()
    scatter_fn = _make_scatter_kernel()
    for l in range(2):
        m0, m1 = msg_fn(a_proj, b_proj, cs[l], dst, src,
                        p['g_msg%d' % l], p['be_msg%d' % l])
        ag0, ag1 = scatter_fn(m0, m1, dst, zeros)
        if l == 0:
            h, a_proj, b_proj = _update(
                h, ag0, ag1, p['W_upd0'], r2(p['b_upd0']), r2(p['g_upd0']),
                r2(p['be_upd0']), r2(p['g_n0']), r2(p['be_n0']),
                w1[1], r2(p['b_msg1']), w2[1])
        else:
            h = _update(
                h, ag0, ag1, p['W_upd1'], r2(p['b_upd1']), r2(p['g_upd1']),
                r2(p['be_upd1']), r2(p['g_n1']), r2(p['be_n1']))

    return _pool_head(h, batch2d, p['W_h1'], r2(p['b_h1']), r2(p['g_h']),
                      r2(p['be_h']), p['W_h2'], r2(p['b_h2']))
